# baseline (speedup 1.0000x reference)
import functools

import jax
import jax.numpy as jnp
from jax import lax
from jax.experimental import pallas as pl
from jax.experimental.pallas import tpu as pltpu
from jax.experimental.pallas import tpu_sc as plsc

DIM = 64
GAMMA = 12.0
LANES = 16
CHUNK = 128
NUM_CORES = 2
NUM_SUBCORES = 16
NUM_WORKERS = NUM_CORES * NUM_SUBCORES

_PERM_DNUMS = lax.GatherDimensionNumbers(
    offset_dims=(), collapsed_slice_dims=(0,), start_index_map=(0,))


def _permute(v, idx):
    return lax.gather(v, idx[:, None], dimension_numbers=_PERM_DNUMS,
                      slice_sizes=(1,),
                      mode=lax.GatherScatterMode.PROMISE_IN_BOUNDS)


@functools.lru_cache(maxsize=None)
def _make_sc_kernel(batch):
    rows_per_w = batch // NUM_WORKERS
    n_chunks = rows_per_w // CHUNK
    mesh = plsc.VectorSubcoreMesh(core_axis_name="c", subcore_axis_name="s")

    def body(hidx_hbm, ridx_hbm, tidx_hbm, ent_hbm, rel_hbm, wrh_hbm, wrt_hbm,
             out_hbm, hidx_v, ridx_v, tidx_v, h_v, t_v, r_v, wh_v, wt_v,
             sc_v, sem):
        wid = lax.axis_index("s") * NUM_CORES + lax.axis_index("c")
        base = wid * rows_per_w
        for c in range(n_chunks):
            cbase = base + c * CHUNK
            pltpu.sync_copy(hidx_hbm.at[pl.ds(cbase, CHUNK)], hidx_v)
            pltpu.sync_copy(ridx_hbm.at[pl.ds(cbase, CHUNK)], ridx_v)
            pltpu.sync_copy(tidx_hbm.at[pl.ds(cbase, CHUNK)], tidx_v)
            cps = [
                pltpu.async_copy(ent_hbm.at[hidx_v], h_v, sem),
                pltpu.async_copy(ent_hbm.at[tidx_v], t_v, sem),
                pltpu.async_copy(rel_hbm.at[ridx_v], r_v, sem),
                pltpu.async_copy(wrh_hbm.at[ridx_v], wh_v, sem),
                pltpu.async_copy(wrt_hbm.at[ridx_v], wt_v, sem),
            ]
            for cp in cps:
                cp.wait()

            lane = lax.iota(jnp.int32, LANES)

            def grp_body(g, carry):
                svec = jnp.zeros((LANES,), jnp.float32)
                for k in range(LANES):
                    i = g * LANES + k
                    acc = jnp.zeros((LANES,), jnp.float32)
                    for j in range(DIM // LANES):
                        sl = pl.ds(j * LANES, LANES)
                        acc = acc + jnp.abs(
                            wh_v[i, sl] * h_v[i, sl] + r_v[i, sl]
                            - wt_v[i, sl] * t_v[i, sl])
                    for sh in (8, 4, 2, 1):
                        acc = acc + _permute(acc, lane ^ sh)
                    svec = jnp.where(lane == k, acc, svec)
                sc_v[pl.ds(g * LANES, LANES)] = svec - GAMMA
                return carry

            lax.fori_loop(0, CHUNK // LANES, grp_body, 0)
            pltpu.sync_copy(sc_v, out_hbm.at[pl.ds(cbase, CHUNK)])

    return pl.kernel(
        body,
        out_type=jax.ShapeDtypeStruct((batch,), jnp.float32),
        mesh=mesh,
        scratch_types=[
            pltpu.VMEM((CHUNK,), jnp.int32),
            pltpu.VMEM((CHUNK,), jnp.int32),
            pltpu.VMEM((CHUNK,), jnp.int32),
            pltpu.VMEM((CHUNK, DIM), jnp.float32),
            pltpu.VMEM((CHUNK, DIM), jnp.float32),
            pltpu.VMEM((CHUNK, DIM), jnp.float32),
            pltpu.VMEM((CHUNK, DIM), jnp.float32),
            pltpu.VMEM((CHUNK, DIM), jnp.float32),
            pltpu.VMEM((CHUNK,), jnp.float32),
            pltpu.SemaphoreType.DMA,
        ],
        compiler_params=pltpu.CompilerParams(use_tc_tiling_on_sc=False),
    )


@jax.jit
def kernel(pos_sample, ent_embd, rel_embd, wrh, wrt):
    batch = pos_sample.shape[0]
    hidx = pos_sample[:, 0]
    ridx = pos_sample[:, 1]
    tidx = pos_sample[:, 2]
    out = _make_sc_kernel(batch)(hidx, ridx, tidx, ent_embd, rel_embd,
                                 wrh, wrt)
    return out[:, None]


# speedup vs baseline: 1.0352x; 1.0352x over previous
import functools

import jax
import jax.numpy as jnp
from jax import lax
from jax.experimental import pallas as pl
from jax.experimental.pallas import tpu as pltpu
from jax.experimental.pallas import tpu_sc as plsc

DIM = 64
GAMMA = 12.0
LANES = 16
CHUNK = 128
NUM_CORES = 2
NUM_SUBCORES = 16
NUM_WORKERS = NUM_CORES * NUM_SUBCORES

_PERM_DNUMS = lax.GatherDimensionNumbers(
    offset_dims=(), collapsed_slice_dims=(0,), start_index_map=(0,))


def _permute(v, idx):
    return lax.gather(v, idx[:, None], dimension_numbers=_PERM_DNUMS,
                      slice_sizes=(1,),
                      mode=lax.GatherScatterMode.PROMISE_IN_BOUNDS)


@functools.lru_cache(maxsize=None)
def _make_sc_kernel(batch):
    rows_per_w = batch // NUM_WORKERS
    n_chunks = rows_per_w // CHUNK
    mesh = plsc.VectorSubcoreMesh(core_axis_name="c", subcore_axis_name="s")

    def body(hidx_hbm, ridx_hbm, tidx_hbm, ent_hbm, rel_hbm, wrh_hbm, wrt_hbm,
             out_hbm, hidx_v, ridx_v, tidx_v, h_v, t_v, r_v, wh_v, wt_v,
             sc_v, gsem, ssem):
        wid = lax.axis_index("s") * NUM_CORES + lax.axis_index("c")
        base = wid * rows_per_w
        pltpu.sync_copy(hidx_hbm.at[pl.ds(base, rows_per_w)], hidx_v)
        pltpu.sync_copy(ridx_hbm.at[pl.ds(base, rows_per_w)], ridx_v)
        pltpu.sync_copy(tidx_hbm.at[pl.ds(base, rows_per_w)], tidx_v)

        def start(c, s):
            sl = pl.ds(c * CHUNK, CHUNK)
            return [
                pltpu.async_copy(ent_hbm.at[hidx_v.at[sl]], h_v.at[s],
                                 gsem.at[s]),
                pltpu.async_copy(ent_hbm.at[tidx_v.at[sl]], t_v.at[s],
                                 gsem.at[s]),
                pltpu.async_copy(rel_hbm.at[ridx_v.at[sl]], r_v.at[s],
                                 gsem.at[s]),
                pltpu.async_copy(wrh_hbm.at[ridx_v.at[sl]], wh_v.at[s],
                                 gsem.at[s]),
                pltpu.async_copy(wrt_hbm.at[ridx_v.at[sl]], wt_v.at[s],
                                 gsem.at[s]),
            ]

        lane = lax.iota(jnp.int32, LANES)
        cps = {0: start(0, 0)}
        st_cps = {}
        for c in range(n_chunks):
            s = c % 2
            if c + 1 < n_chunks:
                cps[c + 1] = start(c + 1, 1 - s)
            for cp in cps.pop(c):
                cp.wait()
            hb, tb, rb, whb, wtb = (h_v.at[s], t_v.at[s], r_v.at[s],
                                    wh_v.at[s], wt_v.at[s])

            def grp_body(g, carry, hb=hb, tb=tb, rb=rb, whb=whb, wtb=wtb,
                         s=s):
                svec = jnp.zeros((LANES,), jnp.float32)
                for k in range(LANES):
                    i = g * LANES + k
                    acc = jnp.zeros((LANES,), jnp.float32)
                    for j in range(DIM // LANES):
                        sl2 = pl.ds(j * LANES, LANES)
                        acc = acc + jnp.abs(
                            whb[i, sl2] * hb[i, sl2] + rb[i, sl2]
                            - wtb[i, sl2] * tb[i, sl2])
                    for sh in (8, 4, 2, 1):
                        acc = acc + _permute(acc, lane ^ sh)
                    svec = jnp.where(lane == k, acc, svec)
                sc_v[s, pl.ds(g * LANES, LANES)] = svec - GAMMA
                return carry

            lax.fori_loop(0, CHUNK // LANES, grp_body, 0)
            if c >= 2:
                st_cps.pop(c - 2).wait()
            st_cps[c] = pltpu.async_copy(
                sc_v.at[s], out_hbm.at[pl.ds(base + c * CHUNK, CHUNK)],
                ssem.at[s])
        for c in list(st_cps):
            st_cps.pop(c).wait()

    return pl.kernel(
        body,
        out_type=jax.ShapeDtypeStruct((batch,), jnp.float32),
        mesh=mesh,
        scratch_types=[
            pltpu.VMEM((rows_per_w,), jnp.int32),
            pltpu.VMEM((rows_per_w,), jnp.int32),
            pltpu.VMEM((rows_per_w,), jnp.int32),
            pltpu.VMEM((2, CHUNK, DIM), jnp.float32),
            pltpu.VMEM((2, CHUNK, DIM), jnp.float32),
            pltpu.VMEM((2, CHUNK, DIM), jnp.float32),
            pltpu.VMEM((2, CHUNK, DIM), jnp.float32),
            pltpu.VMEM((2, CHUNK, DIM), jnp.float32),
            pltpu.VMEM((2, CHUNK), jnp.float32),
            pltpu.SemaphoreType.DMA((2,)),
            pltpu.SemaphoreType.DMA((2,)),
        ],
        compiler_params=pltpu.CompilerParams(use_tc_tiling_on_sc=False),
    )


@jax.jit
def kernel(pos_sample, ent_embd, rel_embd, wrh, wrt):
    batch = pos_sample.shape[0]
    out = _make_sc_kernel(batch)(pos_sample[:, 0], pos_sample[:, 1],
                                 pos_sample[:, 2], ent_embd, rel_embd,
                                 wrh, wrt)
    return out[:, None]
